# baseline (speedup 1.0000x reference)
import functools

import jax
import jax.numpy as jnp
from jax import lax
from jax.experimental import pallas as pl
from jax.experimental.pallas import tpu as pltpu
from jax.experimental.pallas import tpu_sc as plsc

N_NEURONS = 1000
EMBED_DIM = 128
BATCH = 1024
N_SELECTED = 16384

NC, NS, L = 2, 16, 16
NW = NC * NS
B_PER_W = N_SELECTED // NW
AUX_F = 32
AUX_B = AUX_F * 4
HALF = B_PER_W // 2


def _worker_base():
    wid = lax.axis_index("s") * NC + lax.axis_index("c")
    return wid * B_PER_W


NSTREAM = 4
SCHUNK = B_PER_W // NSTREAM


def _sc_aux_body(scale_hbm, bias_hbm, keys_hbm, sg_hbm, bg_hbm,
                 idx_v, s_v, b_v, s_sh, b_sh, sem0, sem1):
    base = _worker_base()

    @pl.when(lax.axis_index("s") == 0)
    def _():
        pltpu.sync_copy(scale_hbm, s_sh)
        pltpu.sync_copy(bias_hbm, b_sh)

    pltpu.sync_copy(keys_hbm.at[pl.ds(base, B_PER_W)], idx_v)
    plsc.subcore_barrier()
    cp0 = pltpu.async_copy(s_sh.at[idx_v], s_v, sem0)
    cp1 = pltpu.async_copy(b_sh.at[idx_v], b_v, sem1)
    cp0.wait()
    pltpu.sync_copy(s_v, sg_hbm.at[pl.ds(base, B_PER_W)])
    cp1.wait()
    pltpu.sync_copy(b_v, bg_hbm.at[pl.ds(base, B_PER_W)])


def _sc_emb_body(table_hbm, keys_hbm, emb_hbm, idx_v, rows_v, tab_sh, sem):
    base = _worker_base()

    @pl.when(lax.axis_index("s") == 0)
    def _():
        pltpu.sync_copy(table_hbm, tab_sh)

    pltpu.sync_copy(keys_hbm.at[pl.ds(base, B_PER_W)], idx_v)
    plsc.subcore_barrier()
    pltpu.async_copy(tab_sh.at[idx_v], rows_v, sem).wait()
    pltpu.sync_copy(rows_v, emb_hbm.at[pl.ds(base, B_PER_W)])


def _sc_mesh():
    return plsc.VectorSubcoreMesh(core_axis_name="c", subcore_axis_name="s",
                                  num_cores=NC, num_subcores=NS)


@functools.cache
def _sc_aux():
    return pl.kernel(
        _sc_aux_body,
        out_type=(
            jax.ShapeDtypeStruct((N_SELECTED,), jnp.float32),
            jax.ShapeDtypeStruct((N_SELECTED,), jnp.float32),
        ),
        mesh=_sc_mesh(),
        scratch_types=[
            pltpu.VMEM((B_PER_W,), jnp.int32),
            pltpu.VMEM((B_PER_W,), jnp.float32),
            pltpu.VMEM((B_PER_W,), jnp.float32),
            pltpu.VMEM_SHARED((N_NEURONS,), jnp.float32),
            pltpu.VMEM_SHARED((N_NEURONS,), jnp.float32),
            pltpu.SemaphoreType.DMA,
            pltpu.SemaphoreType.DMA,
        ],
    )


@functools.cache
def _sc_emb():
    return pl.kernel(
        _sc_emb_body,
        out_type=jax.ShapeDtypeStruct((N_SELECTED, EMBED_DIM), jnp.float32),
        mesh=_sc_mesh(),
        scratch_types=[
            pltpu.VMEM((B_PER_W,), jnp.int32),
            pltpu.VMEM((B_PER_W, EMBED_DIM), jnp.float32),
            pltpu.VMEM_SHARED((N_NEURONS, EMBED_DIM), jnp.float32),
            pltpu.SemaphoreType.DMA,
        ],
    )


def _tc_affine_body(x_ref, s_ref, b_ref, o_ref):
    o_ref[...] = b_ref[...] + s_ref[...] * x_ref[...]


ROW_BLK = 128
COL_BLK = N_SELECTED

_tc_affine = pl.pallas_call(
    _tc_affine_body,
    grid=(BATCH // ROW_BLK,),
    in_specs=[
        pl.BlockSpec((ROW_BLK, COL_BLK), lambda i: (i, 0)),
        pl.BlockSpec((1, COL_BLK), lambda i: (0, 0)),
        pl.BlockSpec((1, COL_BLK), lambda i: (0, 0)),
    ],
    out_specs=pl.BlockSpec((ROW_BLK, COL_BLK), lambda i: (i, 0)),
    out_shape=jax.ShapeDtypeStruct((BATCH, N_SELECTED), jnp.float32),
)


def kernel(inputs, input_axon_embeddings, scale, bias, keys_idx):
    keys32 = keys_idx.astype(jnp.int32)
    sg, bg = _sc_aux()(scale, bias, keys32)
    out_emb = _sc_emb()(input_axon_embeddings, keys32)
    scale_g = sg.reshape(1, N_SELECTED)
    bias_g = bg.reshape(1, N_SELECTED)
    out_inputs = _tc_affine(inputs, scale_g, bias_g)
    return (out_inputs, out_emb)


# speedup vs baseline: 1.0059x; 1.0059x over previous
import functools

import jax
import jax.numpy as jnp
from jax import lax
from jax.experimental import pallas as pl
from jax.experimental.pallas import tpu as pltpu
from jax.experimental.pallas import tpu_sc as plsc

N_NEURONS = 1000
EMBED_DIM = 128
BATCH = 1024
N_SELECTED = 16384

NC, NS, L = 2, 16, 16
NW = NC * NS
B_PER_W = N_SELECTED // NW


def _worker_base():
    wid = lax.axis_index("s") * NC + lax.axis_index("c")
    return wid * B_PER_W


def _sc_aux_body(scale_hbm, bias_hbm, keys_hbm, sg_hbm, bg_hbm,
                 idx_v, s_v, b_v, s_sh, b_sh, sem0, sem1):
    base = _worker_base()

    @pl.when(lax.axis_index("s") == 0)
    def _():
        pltpu.sync_copy(scale_hbm, s_sh)
        pltpu.sync_copy(bias_hbm, b_sh)

    pltpu.sync_copy(keys_hbm.at[pl.ds(base, B_PER_W)], idx_v)
    plsc.subcore_barrier()
    cp0 = pltpu.async_copy(s_sh.at[idx_v], s_v, sem0)
    cp1 = pltpu.async_copy(b_sh.at[idx_v], b_v, sem1)
    cp0.wait()
    pltpu.sync_copy(s_v, sg_hbm.at[pl.ds(base, B_PER_W)])
    cp1.wait()
    pltpu.sync_copy(b_v, bg_hbm.at[pl.ds(base, B_PER_W)])


def _sc_emb_body(table_hbm, keys_hbm, emb_hbm, idx_v, rows_v, tab_sh, sem):
    base = _worker_base()

    @pl.when(lax.axis_index("s") == 0)
    def _():
        pltpu.sync_copy(table_hbm, tab_sh)

    pltpu.sync_copy(keys_hbm.at[pl.ds(base, B_PER_W)], idx_v)
    plsc.subcore_barrier()
    pltpu.async_copy(tab_sh.at[idx_v], rows_v, sem).wait()
    pltpu.sync_copy(rows_v, emb_hbm.at[pl.ds(base, B_PER_W)])


def _sc_mesh():
    return plsc.VectorSubcoreMesh(core_axis_name="c", subcore_axis_name="s",
                                  num_cores=NC, num_subcores=NS)


@functools.cache
def _sc_aux():
    return pl.kernel(
        _sc_aux_body,
        out_type=(
            jax.ShapeDtypeStruct((N_SELECTED,), jnp.float32),
            jax.ShapeDtypeStruct((N_SELECTED,), jnp.float32),
        ),
        mesh=_sc_mesh(),
        scratch_types=[
            pltpu.VMEM((B_PER_W,), jnp.int32),
            pltpu.VMEM((B_PER_W,), jnp.float32),
            pltpu.VMEM((B_PER_W,), jnp.float32),
            pltpu.VMEM_SHARED((N_NEURONS,), jnp.float32),
            pltpu.VMEM_SHARED((N_NEURONS,), jnp.float32),
            pltpu.SemaphoreType.DMA,
            pltpu.SemaphoreType.DMA,
        ],
    )


@functools.cache
def _sc_emb():
    return pl.kernel(
        _sc_emb_body,
        out_type=jax.ShapeDtypeStruct((N_SELECTED, EMBED_DIM), jnp.float32),
        mesh=_sc_mesh(),
        scratch_types=[
            pltpu.VMEM((B_PER_W,), jnp.int32),
            pltpu.VMEM((B_PER_W, EMBED_DIM), jnp.float32),
            pltpu.VMEM_SHARED((N_NEURONS, EMBED_DIM), jnp.float32),
            pltpu.SemaphoreType.DMA,
        ],
    )


def _tc_affine_body(x_ref, s_ref, b_ref, o_ref):
    o_ref[...] = b_ref[...] + s_ref[...] * x_ref[...]


ROW_BLK = 512
COL_BLK = 4096

_tc_affine = pl.pallas_call(
    _tc_affine_body,
    grid=(BATCH // ROW_BLK, N_SELECTED // COL_BLK),
    in_specs=[
        pl.BlockSpec((ROW_BLK, COL_BLK), lambda i, j: (i, j)),
        pl.BlockSpec((1, COL_BLK), lambda i, j: (0, j)),
        pl.BlockSpec((1, COL_BLK), lambda i, j: (0, j)),
    ],
    out_specs=pl.BlockSpec((ROW_BLK, COL_BLK), lambda i, j: (i, j)),
    out_shape=jax.ShapeDtypeStruct((BATCH, N_SELECTED), jnp.float32),
)


def kernel(inputs, input_axon_embeddings, scale, bias, keys_idx):
    keys32 = keys_idx.astype(jnp.int32)
    sg, bg = _sc_aux()(scale, bias, keys32)
    out_emb = _sc_emb()(input_axon_embeddings, keys32)
    scale_g = sg.reshape(1, N_SELECTED)
    bias_g = bg.reshape(1, N_SELECTED)
    out_inputs = _tc_affine(inputs, scale_g, bias_g)
    return (out_inputs, out_emb)
